# baseline (speedup 1.0000x reference)
import functools

import jax
import jax.numpy as jnp
from jax import lax
from jax.experimental import pallas as pl
from jax.experimental.pallas import tpu as pltpu
from jax.experimental.pallas import tpu_sc as plsc

N = 10000
E = 160000
IN_DIM = 16
OUT_DIM = 16
EDGE_DIM = 16

NC = 2
NS = 16
NW = NC * NS
E_PAD = 163840
EH = E_PAD // 2
EPW = EH // NW
HSTR = EPW // 2
N_SP = 10048
ZROWS = N_SP // NS
OROWS = N // NS

PK = 128 // IN_DIM
E8 = E // PK
EH8 = EH // PK
N8 = N // PK
KD = PK * IN_DIM * OUT_DIM


def _mesh():
    return plsc.VectorSubcoreMesh(
        core_axis_name="c", subcore_axis_name="s", num_cores=NC, num_subcores=NS
    )


_SC_PARAMS = pltpu.CompilerParams(use_tc_tiling_on_sc=False)


@functools.lru_cache(maxsize=None)
def _sc_gather(off):
    def body(node_hbm, idx_hbm, out_hbm, idx_v, rows_v, sem):
        c = lax.axis_index("c")
        s = lax.axis_index("s")
        wid = s * NC + c
        base = wid * EPW
        pltpu.sync_copy(idx_hbm.at[pl.ds(off + base, EPW)], idx_v)
        pltpu.async_copy(node_hbm.at[idx_v], rows_v, sem).wait()
        pltpu.sync_copy(rows_v, out_hbm.at[pl.ds(base, EPW)])

    return pl.kernel(
        body,
        out_type=jax.ShapeDtypeStruct((EH, IN_DIM), jnp.float32),
        mesh=_mesh(),
        compiler_params=_SC_PARAMS,
        scratch_types=[
            pltpu.VMEM((EPW,), jnp.int32),
            pltpu.VMEM((EPW, IN_DIM), jnp.float32),
            pltpu.SemaphoreType.DMA,
        ],
    )


@functools.lru_cache(maxsize=None)
def _sc_scatter(off):
    def body(msg_hbm, idx_hbm, zeros_hbm, ones_hbm, agg_out, cnt_out,
             idx_a, idx_b, val_v, ones_v, agg_sh, cnt_sh):
        c = lax.axis_index("c")
        s = lax.axis_index("s")
        wid = s * NC + c
        base = wid * EPW

        pltpu.sync_copy(zeros_hbm, agg_sh.at[pl.ds(s * ZROWS, ZROWS)])
        pltpu.sync_copy(zeros_hbm, cnt_sh.at[pl.ds(s * ZROWS, ZROWS)])
        pltpu.sync_copy(idx_hbm.at[pl.ds(off + base, HSTR)], idx_a)
        pltpu.sync_copy(idx_hbm.at[pl.ds(off + base + HSTR, HSTR)], idx_b)
        pltpu.sync_copy(ones_hbm, ones_v)
        plsc.subcore_barrier()

        pltpu.sync_copy(msg_hbm.at[pl.ds(base, HSTR)], val_v)
        pltpu.sync_copy(val_v, agg_sh.at[idx_a], add=True)
        pltpu.sync_copy(msg_hbm.at[pl.ds(base + HSTR, HSTR)], val_v)
        pltpu.sync_copy(val_v, agg_sh.at[idx_b], add=True)
        pltpu.sync_copy(ones_v, cnt_sh.at[idx_a], add=True)
        pltpu.sync_copy(ones_v, cnt_sh.at[idx_b], add=True)
        plsc.subcore_barrier()

        pltpu.sync_copy(agg_sh.at[pl.ds(s * OROWS, OROWS)],
                        agg_out.at[c, pl.ds(s * OROWS, OROWS)])
        pltpu.sync_copy(cnt_sh.at[pl.ds(s * OROWS, OROWS)],
                        cnt_out.at[c, pl.ds(s * OROWS, OROWS)])

    return pl.kernel(
        body,
        out_type=(
            jax.ShapeDtypeStruct((NC, N, OUT_DIM), jnp.float32),
            jax.ShapeDtypeStruct((NC, N, OUT_DIM), jnp.float32),
        ),
        mesh=_mesh(),
        compiler_params=_SC_PARAMS,
        scratch_types=[
            pltpu.VMEM((HSTR,), jnp.int32),
            pltpu.VMEM((HSTR,), jnp.int32),
            pltpu.VMEM((HSTR, OUT_DIM), jnp.float32),
            pltpu.VMEM((HSTR, OUT_DIM), jnp.float32),
            pltpu.VMEM_SHARED((N_SP, OUT_DIM), jnp.float32),
            pltpu.VMEM_SHARED((N_SP, OUT_DIM), jnp.float32),
        ],
    )


def _msg_body(ef_ref, x_ref, w2_ref, b2_ref, r2_ref, s2_ref, out_ref):
    ef = ef_ref[...]
    x = x_ref[...]
    t = jnp.tanh(
        jnp.dot(ef, w2_ref[...], preferred_element_type=jnp.float32) + b2_ref[...]
    )
    xb = jnp.dot(x, r2_ref[...], preferred_element_type=jnp.float32)
    out_ref[...] = jnp.dot(xb * t, s2_ref[...], preferred_element_type=jnp.float32)


def _msg_call(ef_pk, x_pk, w2, b2, r2, s2, blk, nblk, ef_off):
    return pl.pallas_call(
        _msg_body,
        grid=(nblk,),
        in_specs=[
            pl.BlockSpec((blk, 128), lambda i: (ef_off + i, 0)),
            pl.BlockSpec((blk, 128), lambda i: (i, 0)),
            pl.BlockSpec((128, KD), lambda i: (0, 0)),
            pl.BlockSpec((1, KD), lambda i: (0, 0)),
            pl.BlockSpec((128, KD), lambda i: (0, 0)),
            pl.BlockSpec((KD, 128), lambda i: (0, 0)),
        ],
        out_specs=pl.BlockSpec((blk, 128), lambda i: (i, 0)),
        out_shape=jax.ShapeDtypeStruct((EH8, 128), jnp.float32),
    )(ef_pk, x_pk, w2, b2, r2, s2)


def _final_body(nf_ref, agg_a_ref, cnt_a_ref, agg_b_ref, cnt_b_ref,
                wr2_ref, m_ref, rb_ref, g_ref, b_ref, out_ref):
    nf = nf_ref[...]
    agg = agg_a_ref[0] + agg_a_ref[1] + agg_b_ref[0] + agg_b_ref[1]
    cnt = cnt_a_ref[0] + cnt_a_ref[1] + cnt_b_ref[0] + cnt_b_ref[1]
    agg = agg / jnp.maximum(cnt, 1.0)
    pre = (
        jnp.dot(nf, wr2_ref[...], preferred_element_type=jnp.float32)
        + agg
        + rb_ref[...]
    )
    csum = jnp.sum(pre, axis=0, keepdims=True)
    csq = jnp.sum(pre * pre, axis=0, keepdims=True)
    mu = jnp.dot(csum, m_ref[...], preferred_element_type=jnp.float32) / N
    musq = jnp.dot(csq, m_ref[...], preferred_element_type=jnp.float32) / N
    var = musq - mu * mu
    out = (pre - mu) / jnp.sqrt(var + 1e-5) * g_ref[...] + b_ref[...]
    out_ref[...] = jnp.where(out >= 0.0, out, 0.01 * out)


def _final_call(nf_pk, agg_a, cnt_a, agg_b, cnt_b, wr2, m, rb, g, b):
    return pl.pallas_call(
        _final_body,
        out_shape=jax.ShapeDtypeStruct((N8, 128), jnp.float32),
    )(nf_pk, agg_a, cnt_a, agg_b, cnt_b, wr2, m, rb, g, b)


def kernel(node_feat, edge_feat, edge_index, batch_index,
           num_sampled_nodes_per_hop, num_sampled_edges_per_hop,
           W_net, b_net, W_root, root_bias, bn_gamma, bn_beta):
    src = edge_index[0]
    dst = edge_index[1]
    pad = E_PAD - E
    src_p = jnp.pad(src, (0, pad))
    dst_p = jnp.pad(dst, (0, pad), constant_values=N)

    eye8 = jnp.eye(PK, dtype=jnp.float32)
    w2 = jnp.kron(eye8, W_net)
    b2 = jnp.tile(b_net, PK).reshape(1, KD)
    k0 = lax.broadcasted_iota(jnp.int32, (128, KD), 0)
    c0 = lax.broadcasted_iota(jnp.int32, (128, KD), 1)
    r2 = ((k0 // IN_DIM == c0 // (IN_DIM * OUT_DIM))
          & (k0 % IN_DIM == (c0 % (IN_DIM * OUT_DIM)) // OUT_DIM)
          ).astype(jnp.float32)
    s0 = lax.broadcasted_iota(jnp.int32, (KD, 128), 0)
    s1 = lax.broadcasted_iota(jnp.int32, (KD, 128), 1)
    s2 = ((s0 // (IN_DIM * OUT_DIM) == s1 // OUT_DIM)
          & (s0 % OUT_DIM == s1 % OUT_DIM)).astype(jnp.float32)

    ef_pk = jnp.stack(
        [edge_feat[a::PK, j] for a in range(PK) for j in range(EDGE_DIM)],
        axis=1,
    )
    zeros = jnp.zeros((ZROWS, OUT_DIM), jnp.float32)
    ones = jnp.ones((HSTR, OUT_DIM), jnp.float32)

    x_a = _sc_gather(0)(node_feat, src_p)
    x_b = _sc_gather(EH)(node_feat, src_p)
    msg_a = _msg_call(ef_pk, x_a.reshape(EH8, 128), w2, b2, r2, s2,
                      320, 32, 0)
    agg_a, cnt_a = _sc_scatter(0)(
        msg_a.reshape(EH, OUT_DIM), dst_p, zeros, ones)
    msg_b = _msg_call(ef_pk, x_b.reshape(EH8, 128), w2, b2, r2, s2,
                      160, 61, 64)
    agg_b, cnt_b = _sc_scatter(EH)(
        msg_b.reshape(EH, OUT_DIM), dst_p, zeros, ones)

    wr2 = jnp.kron(eye8, W_root)
    m0 = lax.broadcasted_iota(jnp.int32, (128, 128), 0)
    m1 = lax.broadcasted_iota(jnp.int32, (128, 128), 1)
    m = (m0 % OUT_DIM == m1 % OUT_DIM).astype(jnp.float32)
    out = _final_call(
        node_feat.reshape(N8, 128), agg_a.reshape(NC, N8, 128),
        cnt_a.reshape(NC, N8, 128), agg_b.reshape(NC, N8, 128),
        cnt_b.reshape(NC, N8, 128), wr2, m,
        jnp.tile(root_bias, PK).reshape(1, 128),
        jnp.tile(bn_gamma, PK).reshape(1, 128),
        jnp.tile(bn_beta, PK).reshape(1, 128),
    )
    return (out.reshape(N, OUT_DIM), edge_index, edge_feat)


# speedup vs baseline: 9.9871x; 9.9871x over previous
import functools

import jax
import jax.numpy as jnp
from jax import lax
from jax.experimental import pallas as pl
from jax.experimental.pallas import tpu as pltpu
from jax.experimental.pallas import tpu_sc as plsc

N = 10000
E = 160000
IN_DIM = 16
OUT_DIM = 16
EDGE_DIM = 16

NC = 2
NS = 16
NW = NC * NS
E_PAD = 163840
EH = E_PAD // 2
EPW = EH // NW
HSTR = EPW // 2
N_SP = 10048
ZROWS = N_SP // NS
OROWS = N // NS

PK = 128 // IN_DIM
E8 = E // PK
EH8 = EH // PK
N8 = N // PK
KD = PK * IN_DIM * OUT_DIM


def _mesh():
    return plsc.VectorSubcoreMesh(
        core_axis_name="c", subcore_axis_name="s", num_cores=NC, num_subcores=NS
    )


_SC_PARAMS = pltpu.CompilerParams(use_tc_tiling_on_sc=False)


@functools.lru_cache(maxsize=None)
def _sc_gather(off):
    def body(node_hbm, idx_hbm, out_hbm, idx_v, rows_v, sem):
        c = lax.axis_index("c")
        s = lax.axis_index("s")
        wid = s * NC + c
        base = wid * EPW
        pltpu.sync_copy(idx_hbm.at[pl.ds(off + base, EPW)], idx_v)
        pltpu.async_copy(node_hbm.at[idx_v], rows_v, sem).wait()
        pltpu.sync_copy(rows_v, out_hbm.at[pl.ds(base, EPW)])

    return pl.kernel(
        body,
        out_type=jax.ShapeDtypeStruct((EH, IN_DIM), jnp.float32),
        mesh=_mesh(),
        compiler_params=_SC_PARAMS,
        scratch_types=[
            pltpu.VMEM((EPW,), jnp.int32),
            pltpu.VMEM((EPW, IN_DIM), jnp.float32),
            pltpu.SemaphoreType.DMA,
        ],
    )


@functools.lru_cache(maxsize=None)
def _sc_scatter(off):
    def body(msg_hbm, idx_hbm, zeros_hbm, ones_hbm, agg_out, cnt_out,
             idx_a, idx_b, val_v, ones_v, agg_sh, cnt_sh):
        c = lax.axis_index("c")
        s = lax.axis_index("s")
        wid = s * NC + c
        base = wid * EPW

        pltpu.sync_copy(zeros_hbm, agg_sh.at[pl.ds(s * ZROWS, ZROWS)])
        pltpu.sync_copy(zeros_hbm, cnt_sh.at[pl.ds(s * ZROWS, ZROWS)])
        pltpu.sync_copy(idx_hbm.at[pl.ds(off + base, HSTR)], idx_a)
        pltpu.sync_copy(idx_hbm.at[pl.ds(off + base + HSTR, HSTR)], idx_b)
        pltpu.sync_copy(ones_hbm, ones_v)
        plsc.subcore_barrier()

        pltpu.sync_copy(msg_hbm.at[pl.ds(base, HSTR)], val_v)
        pltpu.sync_copy(val_v, agg_sh.at[idx_a], add=True)
        pltpu.sync_copy(msg_hbm.at[pl.ds(base + HSTR, HSTR)], val_v)
        pltpu.sync_copy(val_v, agg_sh.at[idx_b], add=True)
        pltpu.sync_copy(ones_v, cnt_sh.at[idx_a], add=True)
        pltpu.sync_copy(ones_v, cnt_sh.at[idx_b], add=True)
        plsc.subcore_barrier()

        pltpu.sync_copy(agg_sh.at[pl.ds(s * OROWS, OROWS)],
                        agg_out.at[c, pl.ds(s * OROWS, OROWS)])
        pltpu.sync_copy(cnt_sh.at[pl.ds(s * OROWS, OROWS)],
                        cnt_out.at[c, pl.ds(s * OROWS, OROWS)])

    return pl.kernel(
        body,
        out_type=(
            jax.ShapeDtypeStruct((NC, N, OUT_DIM), jnp.float32),
            jax.ShapeDtypeStruct((NC, N, OUT_DIM), jnp.float32),
        ),
        mesh=_mesh(),
        compiler_params=_SC_PARAMS,
        scratch_types=[
            pltpu.VMEM((HSTR,), jnp.int32),
            pltpu.VMEM((HSTR,), jnp.int32),
            pltpu.VMEM((HSTR, OUT_DIM), jnp.float32),
            pltpu.VMEM((HSTR, OUT_DIM), jnp.float32),
            pltpu.VMEM_SHARED((N_SP, OUT_DIM), jnp.float32),
            pltpu.VMEM_SHARED((N_SP, OUT_DIM), jnp.float32),
        ],
    )


def _msg_body(ef_ref, x_ref, w2_ref, b2_ref, r2_ref, s2_ref, out_ref):
    ef = ef_ref[...]
    x = x_ref[...]
    t = jnp.tanh(
        jnp.dot(ef, w2_ref[...], preferred_element_type=jnp.float32) + b2_ref[...]
    )
    xb = jnp.dot(x, r2_ref[...], preferred_element_type=jnp.float32)
    out_ref[...] = jnp.dot(xb * t, s2_ref[...], preferred_element_type=jnp.float32)


def _msg_call(ef_pk, x_pk, w2, b2, r2, s2, blk, nblk, ef_off):
    return pl.pallas_call(
        _msg_body,
        grid=(nblk,),
        in_specs=[
            pl.BlockSpec((blk, 128), lambda i: (ef_off + i, 0)),
            pl.BlockSpec((blk, 128), lambda i: (i, 0)),
            pl.BlockSpec((128, KD), lambda i: (0, 0)),
            pl.BlockSpec((1, KD), lambda i: (0, 0)),
            pl.BlockSpec((128, KD), lambda i: (0, 0)),
            pl.BlockSpec((KD, 128), lambda i: (0, 0)),
        ],
        out_specs=pl.BlockSpec((blk, 128), lambda i: (i, 0)),
        out_shape=jax.ShapeDtypeStruct((EH8, 128), jnp.float32),
    )(ef_pk, x_pk, w2, b2, r2, s2)


def _final_body(nf_ref, agg_a_ref, cnt_a_ref, agg_b_ref, cnt_b_ref,
                wr2_ref, m_ref, rb_ref, g_ref, b_ref, out_ref):
    nf = nf_ref[...]
    agg = agg_a_ref[0] + agg_a_ref[1] + agg_b_ref[0] + agg_b_ref[1]
    cnt = cnt_a_ref[0] + cnt_a_ref[1] + cnt_b_ref[0] + cnt_b_ref[1]
    agg = agg / jnp.maximum(cnt, 1.0)
    pre = (
        jnp.dot(nf, wr2_ref[...], preferred_element_type=jnp.float32)
        + agg
        + rb_ref[...]
    )
    csum = jnp.sum(pre, axis=0, keepdims=True)
    csq = jnp.sum(pre * pre, axis=0, keepdims=True)
    mu = jnp.dot(csum, m_ref[...], preferred_element_type=jnp.float32) / N
    musq = jnp.dot(csq, m_ref[...], preferred_element_type=jnp.float32) / N
    var = musq - mu * mu
    out = (pre - mu) / jnp.sqrt(var + 1e-5) * g_ref[...] + b_ref[...]
    out_ref[...] = jnp.where(out >= 0.0, out, 0.01 * out)


def _final_call(nf_pk, agg_a, cnt_a, agg_b, cnt_b, wr2, m, rb, g, b):
    return pl.pallas_call(
        _final_body,
        out_shape=jax.ShapeDtypeStruct((N8, 128), jnp.float32),
    )(nf_pk, agg_a, cnt_a, agg_b, cnt_b, wr2, m, rb, g, b)


def kernel(node_feat, edge_feat, edge_index, batch_index,
           num_sampled_nodes_per_hop, num_sampled_edges_per_hop,
           W_net, b_net, W_root, root_bias, bn_gamma, bn_beta):
    src = edge_index[0]
    dst = edge_index[1]
    pad = E_PAD - E
    src_p = jnp.pad(src, (0, pad))
    dst_p = jnp.pad(dst, (0, pad), constant_values=N)

    eye8 = jnp.eye(PK, dtype=jnp.float32)
    w2 = jnp.kron(eye8, W_net)
    b2 = jnp.tile(b_net, PK).reshape(1, KD)
    k0 = lax.broadcasted_iota(jnp.int32, (128, KD), 0)
    c0 = lax.broadcasted_iota(jnp.int32, (128, KD), 1)
    r2 = ((k0 // IN_DIM == c0 // (IN_DIM * OUT_DIM))
          & (k0 % IN_DIM == (c0 % (IN_DIM * OUT_DIM)) // OUT_DIM)
          ).astype(jnp.float32)
    s0 = lax.broadcasted_iota(jnp.int32, (KD, 128), 0)
    s1 = lax.broadcasted_iota(jnp.int32, (KD, 128), 1)
    s2 = ((s0 // (IN_DIM * OUT_DIM) == s1 // OUT_DIM)
          & (s0 % OUT_DIM == s1 % OUT_DIM)).astype(jnp.float32)

    ef_pk = jnp.concatenate(
        [edge_feat.reshape(E8, 128),
         jnp.zeros((EH8 * 2 - E8, 128), jnp.float32)])
    zeros = jnp.zeros((ZROWS, OUT_DIM), jnp.float32)
    ones = jnp.ones((HSTR, OUT_DIM), jnp.float32)

    x_a = _sc_gather(0)(node_feat, src_p)
    x_b = _sc_gather(EH)(node_feat, src_p)
    msg_a = _msg_call(ef_pk, x_a.reshape(EH8, 128), w2, b2, r2, s2,
                      512, 20, 0)
    agg_a, cnt_a = _sc_scatter(0)(
        msg_a.reshape(EH, OUT_DIM), dst_p, zeros, ones)
    msg_b = _msg_call(ef_pk, x_b.reshape(EH8, 128), w2, b2, r2, s2,
                      512, 20, 20)
    agg_b, cnt_b = _sc_scatter(EH)(
        msg_b.reshape(EH, OUT_DIM), dst_p, zeros, ones)

    wr2 = jnp.kron(eye8, W_root)
    m0 = lax.broadcasted_iota(jnp.int32, (128, 128), 0)
    m1 = lax.broadcasted_iota(jnp.int32, (128, 128), 1)
    m = (m0 % OUT_DIM == m1 % OUT_DIM).astype(jnp.float32)
    out = _final_call(
        node_feat.reshape(N8, 128), agg_a.reshape(NC, N8, 128),
        cnt_a.reshape(NC, N8, 128), agg_b.reshape(NC, N8, 128),
        cnt_b.reshape(NC, N8, 128), wr2, m,
        jnp.tile(root_bias, PK).reshape(1, 128),
        jnp.tile(bn_gamma, PK).reshape(1, 128),
        jnp.tile(bn_beta, PK).reshape(1, 128),
    )
    return (out.reshape(N, OUT_DIM), edge_index, edge_feat)


# speedup vs baseline: 10.0965x; 1.0110x over previous
import functools

import jax
import jax.numpy as jnp
from jax import lax
from jax.experimental import pallas as pl
from jax.experimental.pallas import tpu as pltpu
from jax.experimental.pallas import tpu_sc as plsc

N = 10000
E = 160000
IN_DIM = 16
OUT_DIM = 16
EDGE_DIM = 16

NC = 2
NS = 16
NW = NC * NS
EPW = 5120
E_PAD = EPW * NW
HALF = EPW // 2
N_SP = 10048
ZROWS = N_SP // NS
OROWS = N // NS

PK = 128 // IN_DIM
E8 = E // PK
E8_PAD = E_PAD // PK
N8 = N // PK
KD = PK * IN_DIM * OUT_DIM


def _sc_gather_body(node_hbm, idx_hbm, out_hbm, idx_v, rows_v, sem):
    c = lax.axis_index("c")
    s = lax.axis_index("s")
    wid = s * NC + c
    base = wid * EPW
    pltpu.sync_copy(idx_hbm.at[pl.ds(base, EPW)], idx_v)
    pltpu.async_copy(node_hbm.at[idx_v], rows_v, sem).wait()
    pltpu.sync_copy(rows_v, out_hbm.at[pl.ds(base, EPW)])


@functools.lru_cache(maxsize=None)
def _sc_gather():
    mesh = plsc.VectorSubcoreMesh(
        core_axis_name="c", subcore_axis_name="s", num_cores=NC, num_subcores=NS
    )
    return pl.kernel(
        _sc_gather_body,
        out_type=jax.ShapeDtypeStruct((E_PAD, IN_DIM), jnp.float32),
        mesh=mesh,
        compiler_params=pltpu.CompilerParams(use_tc_tiling_on_sc=False),
        scratch_types=[
            pltpu.VMEM((EPW,), jnp.int32),
            pltpu.VMEM((EPW, IN_DIM), jnp.float32),
            pltpu.SemaphoreType.DMA,
        ],
    )


def _sc_scatter_body(msg_hbm, idx_hbm, zeros_hbm, ones_hbm, agg_out, cnt_out,
                     idx_a, idx_b, val_v, ones_v, agg_sh, cnt_sh):
    c = lax.axis_index("c")
    s = lax.axis_index("s")
    wid = s * NC + c
    base = wid * EPW

    pltpu.sync_copy(zeros_hbm, agg_sh.at[pl.ds(s * ZROWS, ZROWS)])
    pltpu.sync_copy(zeros_hbm, cnt_sh.at[pl.ds(s * ZROWS, ZROWS)])
    pltpu.sync_copy(idx_hbm.at[pl.ds(base, HALF)], idx_a)
    pltpu.sync_copy(idx_hbm.at[pl.ds(base + HALF, HALF)], idx_b)
    pltpu.sync_copy(ones_hbm, ones_v)
    plsc.subcore_barrier()

    pltpu.sync_copy(msg_hbm.at[pl.ds(base, HALF)], val_v)
    pltpu.sync_copy(val_v, agg_sh.at[idx_a], add=True)
    pltpu.sync_copy(msg_hbm.at[pl.ds(base + HALF, HALF)], val_v)
    pltpu.sync_copy(val_v, agg_sh.at[idx_b], add=True)
    pltpu.sync_copy(ones_v, cnt_sh.at[idx_a], add=True)
    pltpu.sync_copy(ones_v, cnt_sh.at[idx_b], add=True)
    plsc.subcore_barrier()

    pltpu.sync_copy(agg_sh.at[pl.ds(s * OROWS, OROWS)],
                    agg_out.at[c, pl.ds(s * OROWS, OROWS)])
    pltpu.sync_copy(cnt_sh.at[pl.ds(s * OROWS, OROWS)],
                    cnt_out.at[c, pl.ds(s * OROWS, OROWS)])


@functools.lru_cache(maxsize=None)
def _sc_scatter():
    mesh = plsc.VectorSubcoreMesh(
        core_axis_name="c", subcore_axis_name="s", num_cores=NC, num_subcores=NS
    )
    return pl.kernel(
        _sc_scatter_body,
        out_type=(
            jax.ShapeDtypeStruct((NC, N, OUT_DIM), jnp.float32),
            jax.ShapeDtypeStruct((NC, N, OUT_DIM), jnp.float32),
        ),
        mesh=mesh,
        compiler_params=pltpu.CompilerParams(use_tc_tiling_on_sc=False),
        scratch_types=[
            pltpu.VMEM((HALF,), jnp.int32),
            pltpu.VMEM((HALF,), jnp.int32),
            pltpu.VMEM((HALF, OUT_DIM), jnp.float32),
            pltpu.VMEM((HALF, OUT_DIM), jnp.float32),
            pltpu.VMEM_SHARED((N_SP, OUT_DIM), jnp.float32),
            pltpu.VMEM_SHARED((N_SP, OUT_DIM), jnp.float32),
        ],
    )


MSG_BLK8 = 400


def _msg_body(ef_ref, x_ref, w2_ref, b2_ref, r2_ref, s2_ref, out_ref):
    ef = ef_ref[...]
    x = x_ref[...]
    t = jnp.tanh(
        jnp.dot(ef, w2_ref[...], preferred_element_type=jnp.float32) + b2_ref[...]
    )
    xb = jnp.dot(x, r2_ref[...], preferred_element_type=jnp.float32)
    out_ref[...] = jnp.dot(xb * t, s2_ref[...], preferred_element_type=jnp.float32)


def _msg_call(ef_pk, x_pk, w2, b2, r2, s2):
    return pl.pallas_call(
        _msg_body,
        grid=(E8 // MSG_BLK8,),
        in_specs=[
            pl.BlockSpec((MSG_BLK8, 128), lambda i: (i, 0)),
            pl.BlockSpec((MSG_BLK8, 128), lambda i: (i, 0)),
            pl.BlockSpec((128, KD), lambda i: (0, 0)),
            pl.BlockSpec((1, KD), lambda i: (0, 0)),
            pl.BlockSpec((128, KD), lambda i: (0, 0)),
            pl.BlockSpec((KD, 128), lambda i: (0, 0)),
        ],
        out_specs=pl.BlockSpec((MSG_BLK8, 128), lambda i: (i, 0)),
        out_shape=jax.ShapeDtypeStruct((E8_PAD, 128), jnp.float32),
    )(ef_pk, x_pk, w2, b2, r2, s2)


def _final_body(nf_ref, agg_ref, cnt_ref, wr2_ref, m_ref, rb_ref, g_ref, b_ref,
                out_ref):
    nf = nf_ref[...]
    agg = agg_ref[0] + agg_ref[1]
    cnt = cnt_ref[0] + cnt_ref[1]
    agg = agg / jnp.maximum(cnt, 1.0)
    pre = (
        jnp.dot(nf, wr2_ref[...], preferred_element_type=jnp.float32)
        + agg
        + rb_ref[...]
    )
    csum = jnp.sum(pre, axis=0, keepdims=True)
    csq = jnp.sum(pre * pre, axis=0, keepdims=True)
    mu = jnp.dot(csum, m_ref[...], preferred_element_type=jnp.float32) / N
    musq = jnp.dot(csq, m_ref[...], preferred_element_type=jnp.float32) / N
    var = musq - mu * mu
    out = (pre - mu) / jnp.sqrt(var + 1e-5) * g_ref[...] + b_ref[...]
    out_ref[...] = jnp.where(out >= 0.0, out, 0.01 * out)


def _final_call(nf_pk, agg_pk, cnt_pk, wr2, m, rb, g, b):
    return pl.pallas_call(
        _final_body,
        out_shape=jax.ShapeDtypeStruct((N8, 128), jnp.float32),
    )(nf_pk, agg_pk, cnt_pk, wr2, m, rb, g, b)


def kernel(node_feat, edge_feat, edge_index, batch_index,
           num_sampled_nodes_per_hop, num_sampled_edges_per_hop,
           W_net, b_net, W_root, root_bias, bn_gamma, bn_beta):
    src = edge_index[0]
    dst = edge_index[1]
    pad = E_PAD - E
    src_p = jnp.pad(src, (0, pad))
    dst_p = jnp.pad(dst, (0, pad), constant_values=N)

    eye8 = jnp.eye(PK, dtype=jnp.float32)
    w2 = jnp.kron(eye8, W_net)
    b2 = jnp.tile(b_net, PK).reshape(1, KD)
    k0 = lax.broadcasted_iota(jnp.int32, (128, KD), 0)
    c0 = lax.broadcasted_iota(jnp.int32, (128, KD), 1)
    r2 = ((k0 // IN_DIM == c0 // (IN_DIM * OUT_DIM))
          & (k0 % IN_DIM == (c0 % (IN_DIM * OUT_DIM)) // OUT_DIM)
          ).astype(jnp.float32)
    s0 = lax.broadcasted_iota(jnp.int32, (KD, 128), 0)
    s1 = lax.broadcasted_iota(jnp.int32, (KD, 128), 1)
    s2 = ((s0 // (IN_DIM * OUT_DIM) == s1 // OUT_DIM)
          & (s0 % OUT_DIM == s1 % OUT_DIM)).astype(jnp.float32)

    x_src = _sc_gather()(node_feat, src_p)
    msg = _msg_call(edge_feat.reshape(E8, 128), x_src.reshape(E8_PAD, 128),
                    w2, b2, r2, s2)
    agg_parts, cnt_parts = _sc_scatter()(
        msg.reshape(E_PAD, OUT_DIM), dst_p,
        jnp.zeros((ZROWS, OUT_DIM), jnp.float32),
        jnp.ones((HALF, OUT_DIM), jnp.float32),
    )

    wr2 = jnp.kron(eye8, W_root)
    m0 = lax.broadcasted_iota(jnp.int32, (128, 128), 0)
    m1 = lax.broadcasted_iota(jnp.int32, (128, 128), 1)
    m = (m0 % OUT_DIM == m1 % OUT_DIM).astype(jnp.float32)
    out = _final_call(
        node_feat.reshape(N8, 128), agg_parts.reshape(NC, N8, 128),
        cnt_parts.reshape(NC, N8, 128), wr2, m,
        jnp.tile(root_bias, PK).reshape(1, 128),
        jnp.tile(bn_gamma, PK).reshape(1, 128),
        jnp.tile(bn_beta, PK).reshape(1, 128),
    )
    return (out.reshape(N, OUT_DIM), edge_index, edge_feat)
